# baseline (speedup 1.0000x reference)
import jax
import jax.numpy as jnp
from jax import lax
from jax.experimental import pallas as pl
from jax.experimental.pallas import tpu as pltpu
from jax.experimental.pallas import tpu_sc as plsc

B = 8
N = 16384
S = 2048
L = 16
NC = 2
NSUB = 16

B_TC = 4
B_SC = B - B_TC
TPB = (NC * NSUB) // B_SC
NPT = N // TPB
NCHUNK = NPT // L

_INT_MAX = 2**31 - 1


def _bcast_lane(vec, lane):
    return vec[jnp.full((L,), lane, jnp.int32)]


def _fps_sc_body(xyz_hbm, out_hbm, x_v, y_v, z_v, d_v, pub_v, part_v, oidx_v,
                 shared):
    cid = lax.axis_index("c")
    sid = lax.axis_index("s")
    wid = cid * NSUB + sid
    b = wid // TPB
    k = wid % TPB
    grp = sid - (sid % TPB)

    lane_iota = lax.iota(jnp.int32, L)

    base = b * 3 * N + k * NPT
    pltpu.sync_copy(xyz_hbm.at[pl.ds(base, NPT)], x_v)
    pltpu.sync_copy(xyz_hbm.at[pl.ds(base + N, NPT)], y_v)
    pltpu.sync_copy(xyz_hbm.at[pl.ds(base + 2 * N, NPT)], z_v)

    inf_v = jnp.full((L,), jnp.inf, jnp.float32)

    def init_body(c, carry):
        d_v[pl.ds(c * L, L)] = inf_v
        return carry

    lax.fori_loop(0, NCHUNK, init_body, 0)

    cb = b * 3 * N
    pltpu.sync_copy(xyz_hbm.at[pl.ds(cb, L)], pub_v)
    cxv0 = _bcast_lane(pub_v[...], 0)
    pltpu.sync_copy(xyz_hbm.at[pl.ds(cb + N, L)], pub_v)
    cyv0 = _bcast_lane(pub_v[...], 0)
    pltpu.sync_copy(xyz_hbm.at[pl.ds(cb + 2 * N, L)], pub_v)
    czv0 = _bcast_lane(pub_v[...], 0)

    gbase = k * NPT

    def step(i, carry):
        cxv, cyv, czv, farv = carry

        @pl.when(k == 0)
        def _():
            plsc.store_scatter(oidx_v, [jnp.full((L,), i, jnp.int32)],
                               farv, mask=lane_iota == 0)

        def chunk(c, inner):
            bmax, bidx = inner
            off = c * L
            xv = x_v[pl.ds(off, L)]
            yv = y_v[pl.ds(off, L)]
            zv = z_v[pl.ds(off, L)]
            dv = d_v[pl.ds(off, L)]
            dx = xv - cxv
            dy = yv - cyv
            dz = zv - czv
            d = (dx * dx + dy * dy) + dz * dz
            nd = jnp.minimum(dv, d)
            d_v[pl.ds(off, L)] = nd
            upd = nd > bmax
            bmax = jnp.where(upd, nd, bmax)
            bidx = jnp.where(upd, lane_iota + off, bidx)
            return bmax, bidx

        bmax0 = jnp.full((L,), -jnp.inf, jnp.float32)
        bidx0 = jnp.zeros((L,), jnp.int32)
        bmax, bidx = plsc.parallel_loop(
            0, NCHUNK, 1, unroll=8, carry=(bmax0, bidx0))(chunk)

        pmax = bmax
        for sh in (8, 4, 2, 1):
            pmax = jnp.maximum(pmax, pmax[lane_iota ^ sh])
        pidx = jnp.where(bmax == pmax, bidx, jnp.int32(_INT_MAX))
        for sh in (8, 4, 2, 1):
            pidx = jnp.minimum(pidx, pidx[lane_iota ^ sh])

        wx = plsc.load_gather(x_v, [pidx])
        wy = plsc.load_gather(y_v, [pidx])
        wz = plsc.load_gather(z_v, [pidx])

        gidx_f = plsc.bitcast(pidx + gbase, jnp.float32)
        pub = jnp.where(lane_iota == 0, pmax,
              jnp.where(lane_iota == 1, gidx_f,
              jnp.where(lane_iota == 2, wx,
              jnp.where(lane_iota == 3, wy, wz))))
        pub_v[...] = pub

        poff = (i % 2) * (NSUB * L)
        pltpu.sync_copy(pub_v, shared.at[pl.ds(poff + sid * L, L)])
        plsc.subcore_barrier()
        pltpu.sync_copy(shared.at[pl.ds(poff + grp * L, TPB * L)], part_v)

        best = part_v[pl.ds(0, L)]
        bval = _bcast_lane(best, 0)
        for j in range(1, TPB):
            cand = part_v[pl.ds(j * L, L)]
            cval = _bcast_lane(cand, 0)
            take = cval > bval
            best = jnp.where(take, cand, best)
            bval = jnp.where(take, cval, bval)

        nfarv = _bcast_lane(plsc.bitcast(best, jnp.int32), 1)
        ncxv = _bcast_lane(best, 2)
        ncyv = _bcast_lane(best, 3)
        nczv = _bcast_lane(best, 4)
        return ncxv, ncyv, nczv, nfarv

    far0 = jnp.zeros((L,), jnp.int32)
    lax.fori_loop(0, S, step, (cxv0, cyv0, czv0, far0))

    @pl.when(k == 0)
    def _():
        pltpu.sync_copy(oidx_v, out_hbm.at[pl.ds(b * S, S)])


BLK = 512
NBLK = N // BLK


def _fps_tc_body(x_ref, y_ref, z_ref, out_ref, dist_ref):
    iota_blk = lax.broadcasted_iota(jnp.int32, (B_TC, BLK), 1)
    dist_ref[...] = jnp.full((B_TC, N), jnp.inf, jnp.float32)

    def step(i, far_col):
        out_ref[pl.ds(i, 1), :] = far_col.reshape(1, B_TC)

        def blk_cent(blk, acc):
            cxa, cya, cza = acc
            sl = pl.ds(blk * BLK, BLK)
            sel = (iota_blk + blk * BLK) == far_col
            z0 = jnp.float32(0.0)
            cxa = cxa + jnp.sum(jnp.where(sel, x_ref[:, sl], z0), axis=1,
                                keepdims=True)
            cya = cya + jnp.sum(jnp.where(sel, y_ref[:, sl], z0), axis=1,
                                keepdims=True)
            cza = cza + jnp.sum(jnp.where(sel, z_ref[:, sl], z0), axis=1,
                                keepdims=True)
            return cxa, cya, cza

        zc = jnp.zeros((B_TC, 1), jnp.float32)
        cx, cy, cz = lax.fori_loop(0, NBLK, blk_cent, (zc, zc, zc))

        def blk_dist(blk, carry):
            bmax, bidx = carry
            sl = pl.ds(blk * BLK, BLK)
            dx = x_ref[:, sl] - cx
            dy = y_ref[:, sl] - cy
            dz = z_ref[:, sl] - cz
            d = (dx * dx + dy * dy) + dz * dz
            nd = jnp.minimum(dist_ref[:, sl], d)
            dist_ref[:, sl] = nd
            upd = nd > bmax
            bmax = jnp.where(upd, nd, bmax)
            bidx = jnp.where(upd, iota_blk + blk * BLK, bidx)
            return bmax, bidx

        bmax0 = jnp.full((B_TC, BLK), -jnp.inf, jnp.float32)
        bidx0 = jnp.zeros((B_TC, BLK), jnp.int32)
        bmax, bidx = lax.fori_loop(0, NBLK, blk_dist, (bmax0, bidx0))

        mx = jnp.max(bmax, axis=1, keepdims=True)
        idx = jnp.min(jnp.where(bmax == mx, bidx, jnp.int32(_INT_MAX)),
                      axis=1, keepdims=True)
        return idx

    lax.fori_loop(0, S, step, jnp.zeros((B_TC, 1), jnp.int32))


@jax.jit
def kernel(points_xyz):
    xyz_t = points_xyz.transpose(0, 2, 1)
    sc_flat = xyz_t[B_TC:].reshape(-1)

    mesh = plsc.VectorSubcoreMesh(core_axis_name="c", subcore_axis_name="s")
    fps_sc = pl.kernel(
        _fps_sc_body,
        out_type=jax.ShapeDtypeStruct((B_SC * S,), jnp.int32),
        mesh=mesh,
        compiler_params=pltpu.CompilerParams(needs_layout_passes=False),
        scratch_types=[
            pltpu.VMEM((NPT,), jnp.float32),
            pltpu.VMEM((NPT,), jnp.float32),
            pltpu.VMEM((NPT,), jnp.float32),
            pltpu.VMEM((NPT,), jnp.float32),
            pltpu.VMEM((L,), jnp.float32),
            pltpu.VMEM((TPB * L,), jnp.float32),
            pltpu.VMEM((S,), jnp.int32),
            pltpu.VMEM_SHARED((2 * NSUB * L,), jnp.float32),
        ],
    )
    sc_out = fps_sc(sc_flat).reshape(B_SC, S)

    tc_out = pl.pallas_call(
        _fps_tc_body,
        out_shape=jax.ShapeDtypeStruct((S, B_TC), jnp.int32),
        scratch_shapes=[pltpu.VMEM((B_TC, N), jnp.float32)],
    )(xyz_t[:B_TC, 0], xyz_t[:B_TC, 1], xyz_t[:B_TC, 2])

    return jnp.concatenate([tc_out.T, sc_out], axis=0)


# speedup vs baseline: 5.2766x; 5.2766x over previous
import jax
import jax.numpy as jnp
from jax import lax
from jax.experimental import pallas as pl
from jax.experimental.pallas import tpu as pltpu
from jax.experimental.pallas import tpu_sc as plsc

B = 8
N = 16384
S = 2048
L = 16
NC = 2
NSUB = 16

B_TC = 4
B_SC = B - B_TC
TPB = (NC * NSUB) // B_SC
NPT = N // TPB
NCHUNK = NPT // L

_INT_MAX = 2**31 - 1


def _bcast_lane(vec, lane):
    return vec[jnp.full((L,), lane, jnp.int32)]


def _fps_sc_body(xyz_hbm, out_hbm, x_v, y_v, z_v, d_v, pub_v, part_v, oidx_v,
                 shared):
    cid = lax.axis_index("c")
    sid = lax.axis_index("s")
    wid = cid * NSUB + sid
    b = wid // TPB
    k = wid % TPB
    grp = sid - (sid % TPB)

    lane_iota = lax.iota(jnp.int32, L)

    base = b * 3 * N + k * NPT
    pltpu.sync_copy(xyz_hbm.at[pl.ds(base, NPT)], x_v)
    pltpu.sync_copy(xyz_hbm.at[pl.ds(base + N, NPT)], y_v)
    pltpu.sync_copy(xyz_hbm.at[pl.ds(base + 2 * N, NPT)], z_v)

    inf_v = jnp.full((L,), jnp.inf, jnp.float32)

    def init_body(c, carry):
        d_v[pl.ds(c * L, L)] = inf_v
        return carry

    lax.fori_loop(0, NCHUNK, init_body, 0)

    cb = b * 3 * N
    pltpu.sync_copy(xyz_hbm.at[pl.ds(cb, L)], pub_v)
    cxv0 = _bcast_lane(pub_v[...], 0)
    pltpu.sync_copy(xyz_hbm.at[pl.ds(cb + N, L)], pub_v)
    cyv0 = _bcast_lane(pub_v[...], 0)
    pltpu.sync_copy(xyz_hbm.at[pl.ds(cb + 2 * N, L)], pub_v)
    czv0 = _bcast_lane(pub_v[...], 0)

    gbase = k * NPT

    def step(i, carry):
        cxv, cyv, czv, farv = carry

        @pl.when(k == 0)
        def _():
            plsc.store_scatter(oidx_v, [jnp.full((L,), i, jnp.int32)],
                               farv, mask=lane_iota == 0)

        def chunk(c, inner):
            bmax, bidx = inner
            off = c * L
            xv = x_v[pl.ds(off, L)]
            yv = y_v[pl.ds(off, L)]
            zv = z_v[pl.ds(off, L)]
            dv = d_v[pl.ds(off, L)]
            dx = xv - cxv
            dy = yv - cyv
            dz = zv - czv
            d = (dx * dx + dy * dy) + dz * dz
            nd = jnp.minimum(dv, d)
            d_v[pl.ds(off, L)] = nd
            upd = nd > bmax
            bmax = jnp.where(upd, nd, bmax)
            bidx = jnp.where(upd, lane_iota + off, bidx)
            return bmax, bidx

        bmax0 = jnp.full((L,), -jnp.inf, jnp.float32)
        bidx0 = jnp.zeros((L,), jnp.int32)
        bmax, bidx = plsc.parallel_loop(
            0, NCHUNK, 1, unroll=8, carry=(bmax0, bidx0))(chunk)

        pmax = bmax
        for sh in (8, 4, 2, 1):
            pmax = jnp.maximum(pmax, pmax[lane_iota ^ sh])
        pidx = jnp.where(bmax == pmax, bidx, jnp.int32(_INT_MAX))
        for sh in (8, 4, 2, 1):
            pidx = jnp.minimum(pidx, pidx[lane_iota ^ sh])

        wx = plsc.load_gather(x_v, [pidx])
        wy = plsc.load_gather(y_v, [pidx])
        wz = plsc.load_gather(z_v, [pidx])

        gidx_f = plsc.bitcast(pidx + gbase, jnp.float32)
        pub = jnp.where(lane_iota == 0, pmax,
              jnp.where(lane_iota == 1, gidx_f,
              jnp.where(lane_iota == 2, wx,
              jnp.where(lane_iota == 3, wy, wz))))
        pub_v[...] = pub

        poff = (i % 2) * (NSUB * L)
        pltpu.sync_copy(pub_v, shared.at[pl.ds(poff + sid * L, L)])
        plsc.subcore_barrier()
        pltpu.sync_copy(shared.at[pl.ds(poff + grp * L, TPB * L)], part_v)

        best = part_v[pl.ds(0, L)]
        bval = _bcast_lane(best, 0)
        for j in range(1, TPB):
            cand = part_v[pl.ds(j * L, L)]
            cval = _bcast_lane(cand, 0)
            take = cval > bval
            best = jnp.where(take, cand, best)
            bval = jnp.where(take, cval, bval)

        nfarv = _bcast_lane(plsc.bitcast(best, jnp.int32), 1)
        ncxv = _bcast_lane(best, 2)
        ncyv = _bcast_lane(best, 3)
        nczv = _bcast_lane(best, 4)
        return ncxv, ncyv, nczv, nfarv

    far0 = jnp.zeros((L,), jnp.int32)
    lax.fori_loop(0, S, step, (cxv0, cyv0, czv0, far0))

    @pl.when(k == 0)
    def _():
        pltpu.sync_copy(oidx_v, out_hbm.at[pl.ds(b * S, S)])


BLK = 512
NBLK = N // BLK


def _fps_tc_body(x_ref, y_ref, z_ref, out_ref, dist_ref):
    iota_blk = lax.broadcasted_iota(jnp.int32, (B_TC, BLK), 1)
    dist_ref[...] = jnp.full((B_TC, N), jnp.inf, jnp.float32)

    def step(i, far_col):
        out_ref[pl.ds(i, 1), :] = far_col.reshape(1, B_TC)

        z0 = jnp.float32(0.0)
        zc = jnp.zeros((B_TC, 1), jnp.float32)
        cx, cy, cz = zc, zc, zc
        for blk in range(NBLK):
            sl = pl.ds(blk * BLK, BLK)
            sel = (iota_blk + blk * BLK) == far_col
            cx = cx + jnp.sum(jnp.where(sel, x_ref[:, sl], z0), axis=1,
                              keepdims=True)
            cy = cy + jnp.sum(jnp.where(sel, y_ref[:, sl], z0), axis=1,
                              keepdims=True)
            cz = cz + jnp.sum(jnp.where(sel, z_ref[:, sl], z0), axis=1,
                              keepdims=True)

        bmax = jnp.full((B_TC, BLK), -jnp.inf, jnp.float32)
        bidx = jnp.zeros((B_TC, BLK), jnp.int32)
        for blk in range(NBLK):
            sl = pl.ds(blk * BLK, BLK)
            dx = x_ref[:, sl] - cx
            dy = y_ref[:, sl] - cy
            dz = z_ref[:, sl] - cz
            d = (dx * dx + dy * dy) + dz * dz
            nd = jnp.minimum(dist_ref[:, sl], d)
            dist_ref[:, sl] = nd
            upd = nd > bmax
            bmax = jnp.where(upd, nd, bmax)
            bidx = jnp.where(upd, iota_blk + blk * BLK, bidx)

        mx = jnp.max(bmax, axis=1, keepdims=True)
        idx = jnp.min(jnp.where(bmax == mx, bidx, jnp.int32(_INT_MAX)),
                      axis=1, keepdims=True)
        return idx

    lax.fori_loop(0, S, step, jnp.zeros((B_TC, 1), jnp.int32))


@jax.jit
def kernel(points_xyz):
    xyz_t = points_xyz.transpose(0, 2, 1)
    sc_flat = xyz_t[B_TC:].reshape(-1)

    mesh = plsc.VectorSubcoreMesh(core_axis_name="c", subcore_axis_name="s")
    fps_sc = pl.kernel(
        _fps_sc_body,
        out_type=jax.ShapeDtypeStruct((B_SC * S,), jnp.int32),
        mesh=mesh,
        compiler_params=pltpu.CompilerParams(needs_layout_passes=False),
        scratch_types=[
            pltpu.VMEM((NPT,), jnp.float32),
            pltpu.VMEM((NPT,), jnp.float32),
            pltpu.VMEM((NPT,), jnp.float32),
            pltpu.VMEM((NPT,), jnp.float32),
            pltpu.VMEM((L,), jnp.float32),
            pltpu.VMEM((TPB * L,), jnp.float32),
            pltpu.VMEM((S,), jnp.int32),
            pltpu.VMEM_SHARED((2 * NSUB * L,), jnp.float32),
        ],
    )
    sc_out = fps_sc(sc_flat).reshape(B_SC, S)

    tc_out = pl.pallas_call(
        _fps_tc_body,
        out_shape=jax.ShapeDtypeStruct((S, B_TC), jnp.int32),
        scratch_shapes=[pltpu.VMEM((B_TC, N), jnp.float32)],
    )(xyz_t[:B_TC, 0], xyz_t[:B_TC, 1], xyz_t[:B_TC, 2])

    return jnp.concatenate([tc_out.T, sc_out], axis=0)


# speedup vs baseline: 5.9987x; 1.1368x over previous
import jax
import jax.numpy as jnp
from jax import lax
from jax.experimental import pallas as pl
from jax.experimental.pallas import tpu as pltpu
from jax.experimental.pallas import tpu_sc as plsc

B = 8
N = 16384
S = 2048
L = 16
NC = 2
NSUB = 16

B_TC = 4
B_SC = B - B_TC
TPB = (NC * NSUB) // B_SC
NPT = N // TPB
NCHUNK = NPT // L

_INT_MAX = 2**31 - 1


def _bcast_lane(vec, lane):
    return vec[jnp.full((L,), lane, jnp.int32)]


def _fps_sc_body(xyz_hbm, out_hbm, x_v, y_v, z_v, d_v, pub_v, part_v, oidx_v,
                 shared):
    cid = lax.axis_index("c")
    sid = lax.axis_index("s")
    wid = cid * NSUB + sid
    b = wid // TPB
    k = wid % TPB
    grp = sid - (sid % TPB)

    lane_iota = lax.iota(jnp.int32, L)

    base = b * 3 * N + k * NPT
    pltpu.sync_copy(xyz_hbm.at[pl.ds(base, NPT)], x_v)
    pltpu.sync_copy(xyz_hbm.at[pl.ds(base + N, NPT)], y_v)
    pltpu.sync_copy(xyz_hbm.at[pl.ds(base + 2 * N, NPT)], z_v)

    inf_v = jnp.full((L,), jnp.inf, jnp.float32)

    def init_body(c, carry):
        d_v[pl.ds(c * L, L)] = inf_v
        return carry

    lax.fori_loop(0, NCHUNK, init_body, 0)

    cb = b * 3 * N
    pltpu.sync_copy(xyz_hbm.at[pl.ds(cb, L)], pub_v)
    cxv0 = _bcast_lane(pub_v[...], 0)
    pltpu.sync_copy(xyz_hbm.at[pl.ds(cb + N, L)], pub_v)
    cyv0 = _bcast_lane(pub_v[...], 0)
    pltpu.sync_copy(xyz_hbm.at[pl.ds(cb + 2 * N, L)], pub_v)
    czv0 = _bcast_lane(pub_v[...], 0)

    gbase = k * NPT

    def step(i, carry):
        cxv, cyv, czv, farv = carry

        @pl.when(k == 0)
        def _():
            plsc.store_scatter(oidx_v, [jnp.full((L,), i, jnp.int32)],
                               farv, mask=lane_iota == 0)

        def chunk(c, inner):
            bmax, bidx = inner
            off = c * L
            xv = x_v[pl.ds(off, L)]
            yv = y_v[pl.ds(off, L)]
            zv = z_v[pl.ds(off, L)]
            dv = d_v[pl.ds(off, L)]
            dx = xv - cxv
            dy = yv - cyv
            dz = zv - czv
            d = (dx * dx + dy * dy) + dz * dz
            nd = jnp.minimum(dv, d)
            d_v[pl.ds(off, L)] = nd
            upd = nd > bmax
            bmax = jnp.where(upd, nd, bmax)
            bidx = jnp.where(upd, lane_iota + off, bidx)
            return bmax, bidx

        bmax0 = jnp.full((L,), -jnp.inf, jnp.float32)
        bidx0 = jnp.zeros((L,), jnp.int32)
        bmax, bidx = plsc.parallel_loop(
            0, NCHUNK, 1, unroll=8, carry=(bmax0, bidx0))(chunk)

        pmax = bmax
        for sh in (8, 4, 2, 1):
            pmax = jnp.maximum(pmax, pmax[lane_iota ^ sh])
        pidx = jnp.where(bmax == pmax, bidx, jnp.int32(_INT_MAX))
        for sh in (8, 4, 2, 1):
            pidx = jnp.minimum(pidx, pidx[lane_iota ^ sh])

        wx = plsc.load_gather(x_v, [pidx])
        wy = plsc.load_gather(y_v, [pidx])
        wz = plsc.load_gather(z_v, [pidx])

        gidx_f = plsc.bitcast(pidx + gbase, jnp.float32)
        pub = jnp.where(lane_iota == 0, pmax,
              jnp.where(lane_iota == 1, gidx_f,
              jnp.where(lane_iota == 2, wx,
              jnp.where(lane_iota == 3, wy, wz))))
        pub_v[...] = pub

        poff = (i % 2) * (NSUB * L)
        pltpu.sync_copy(pub_v, shared.at[pl.ds(poff + sid * L, L)])
        plsc.subcore_barrier()
        pltpu.sync_copy(shared.at[pl.ds(poff + grp * L, TPB * L)], part_v)

        best = part_v[pl.ds(0, L)]
        bval = _bcast_lane(best, 0)
        for j in range(1, TPB):
            cand = part_v[pl.ds(j * L, L)]
            cval = _bcast_lane(cand, 0)
            take = cval > bval
            best = jnp.where(take, cand, best)
            bval = jnp.where(take, cval, bval)

        nfarv = _bcast_lane(plsc.bitcast(best, jnp.int32), 1)
        ncxv = _bcast_lane(best, 2)
        ncyv = _bcast_lane(best, 3)
        nczv = _bcast_lane(best, 4)
        return ncxv, ncyv, nczv, nfarv

    far0 = jnp.zeros((L,), jnp.int32)
    lax.fori_loop(0, S, step, (cxv0, cyv0, czv0, far0))

    @pl.when(k == 0)
    def _():
        pltpu.sync_copy(oidx_v, out_hbm.at[pl.ds(b * S, S)])


BLK = 512
NBLK = N // BLK


def _fps_tc_body(x_ref, y_ref, z_ref, out_ref, dist_ref):
    iota_blk = lax.broadcasted_iota(jnp.int32, (B_TC, BLK), 1)
    dist_ref[...] = jnp.full((B_TC, N), jnp.inf, jnp.float32)

    def step(i, carry):
        far_col, cx, cy, cz = carry
        out_ref[pl.ds(i, 1), :] = far_col.reshape(1, B_TC)

        bmax = jnp.full((B_TC, BLK), -jnp.inf, jnp.float32)
        bidx = jnp.zeros((B_TC, BLK), jnp.int32)
        zblk = jnp.zeros((B_TC, BLK), jnp.float32)
        bx, by, bz = zblk, zblk, zblk
        for blk in range(NBLK):
            sl = pl.ds(blk * BLK, BLK)
            x = x_ref[:, sl]
            y = y_ref[:, sl]
            z = z_ref[:, sl]
            dx = x - cx
            dy = y - cy
            dz = z - cz
            d = (dx * dx + dy * dy) + dz * dz
            nd = jnp.minimum(dist_ref[:, sl], d)
            dist_ref[:, sl] = nd
            upd = nd > bmax
            bmax = jnp.where(upd, nd, bmax)
            bidx = jnp.where(upd, iota_blk + blk * BLK, bidx)
            bx = jnp.where(upd, x, bx)
            by = jnp.where(upd, y, by)
            bz = jnp.where(upd, z, bz)

        mx = jnp.max(bmax, axis=1, keepdims=True)
        idx = jnp.min(jnp.where(bmax == mx, bidx, jnp.int32(_INT_MAX)),
                      axis=1, keepdims=True)
        sel = bidx == idx
        z0 = jnp.float32(0.0)
        ncx = jnp.sum(jnp.where(sel, bx, z0), axis=1, keepdims=True)
        ncy = jnp.sum(jnp.where(sel, by, z0), axis=1, keepdims=True)
        ncz = jnp.sum(jnp.where(sel, bz, z0), axis=1, keepdims=True)
        return idx, ncx, ncy, ncz

    cx0 = x_ref[:, pl.ds(0, BLK)][:, :1]
    cy0 = y_ref[:, pl.ds(0, BLK)][:, :1]
    cz0 = z_ref[:, pl.ds(0, BLK)][:, :1]
    lax.fori_loop(0, S, step,
                  (jnp.zeros((B_TC, 1), jnp.int32), cx0, cy0, cz0))


@jax.jit
def kernel(points_xyz):
    xyz_t = points_xyz.transpose(0, 2, 1)
    sc_flat = xyz_t[B_TC:].reshape(-1)

    mesh = plsc.VectorSubcoreMesh(core_axis_name="c", subcore_axis_name="s")
    fps_sc = pl.kernel(
        _fps_sc_body,
        out_type=jax.ShapeDtypeStruct((B_SC * S,), jnp.int32),
        mesh=mesh,
        compiler_params=pltpu.CompilerParams(needs_layout_passes=False),
        scratch_types=[
            pltpu.VMEM((NPT,), jnp.float32),
            pltpu.VMEM((NPT,), jnp.float32),
            pltpu.VMEM((NPT,), jnp.float32),
            pltpu.VMEM((NPT,), jnp.float32),
            pltpu.VMEM((L,), jnp.float32),
            pltpu.VMEM((TPB * L,), jnp.float32),
            pltpu.VMEM((S,), jnp.int32),
            pltpu.VMEM_SHARED((2 * NSUB * L,), jnp.float32),
        ],
    )
    sc_out = fps_sc(sc_flat).reshape(B_SC, S)

    tc_out = pl.pallas_call(
        _fps_tc_body,
        out_shape=jax.ShapeDtypeStruct((S, B_TC), jnp.int32),
        scratch_shapes=[pltpu.VMEM((B_TC, N), jnp.float32)],
    )(xyz_t[:B_TC, 0], xyz_t[:B_TC, 1], xyz_t[:B_TC, 2])

    return jnp.concatenate([tc_out.T, sc_out], axis=0)
